# baseline (speedup 1.0000x reference)
import functools

import jax
import jax.numpy as jnp
from jax import lax
from jax.experimental import pallas as pl
from jax.experimental.pallas import tpu as pltpu
from jax.experimental.pallas import tpu_sc as plsc

N = 100000
CH = 128
K = 9
R = 65536
NR = K * R
CW = 32
NCORE = 2
NSUB = 16
NCHUNK = CH // CW
PASSES = NCHUNK // NCORE

IDX_W = 128
NR_ROWS = NR // IDX_W
ROWS_PT = NR_ROWS // NSUB
B_ROWS = 4
NBATCH = ROWS_PT // B_ROWS
N_PAD = 100096
ACC_PT = N_PAD // NSUB
INIT_ROWS = 368

BR = 2000


def _mm_body(feats_ref, w_ref, out_ref):
    k = pl.program_id(1)
    out_ref[0] = jnp.dot(feats_ref[...], w_ref[k],
                         preferred_element_type=jnp.float32
                         ).astype(jnp.bfloat16)


def _transform(feats_bf, weight_bf):
    return pl.pallas_call(
        _mm_body,
        grid=(N // BR, K),
        in_specs=[
            pl.BlockSpec((BR, CH), lambda i, k: (i, 0)),
            pl.BlockSpec((K, CH, CH), lambda i, k: (0, 0, 0)),
        ],
        out_specs=pl.BlockSpec((1, BR, CH), lambda i, k: (k, i, 0)),
        out_shape=jax.ShapeDtypeStruct((K, N, CH), jnp.bfloat16),
    )(feats_bf, weight_bf)


def _sc_body(tflat, gidx_all, orows, bias_in, out_hbm,
             acc, bias_v, init_v, gi_v, or_v, data_v, gsem, ssem):
    cid = lax.axis_index("c")
    sid = lax.axis_index("s")
    rule_base = sid * ROWS_PT

    pltpu.sync_copy(bias_in, bias_v)

    for p in range(PASSES):
        chunk = cid * PASSES + p

        b0 = bias_v[pl.ds((0 * PASSES + p) * CW, CW)]
        b1 = bias_v[pl.ds((1 * PASSES + p) * CW, CW)]
        bias_c = jnp.where(cid == 0, b0, b1)

        def fill_row(i, carry):
            init_v[i] = bias_c
            return carry
        lax.fori_loop(0, INIT_ROWS, fill_row, 0)
        for h in range(ACC_PT // INIT_ROWS):
            pltpu.sync_copy(
                init_v, acc.at[pl.ds(sid * ACC_PT + h * INIT_ROWS, INIT_ROWS)])
        plsc.subcore_barrier()

        def do_batch(row_off, buf):
            pltpu.sync_copy(
                gidx_all.at[chunk].at[pl.ds(row_off, B_ROWS)], gi_v.at[buf])
            pltpu.sync_copy(orows.at[pl.ds(row_off, B_ROWS)], or_v.at[buf])
            for j in range(B_ROWS):
                pltpu.async_copy(
                    tflat.at[gi_v.at[buf].at[j]],
                    data_v.at[buf].at[pl.ds(j * IDX_W, IDX_W)], gsem)

        def fire_scatter(buf):
            pltpu.make_async_copy(
                tflat.at[pl.ds(0, B_ROWS * IDX_W)], data_v.at[buf],
                gsem).wait()
            for j in range(B_ROWS):
                pltpu.async_copy(
                    data_v.at[buf].at[pl.ds(j * IDX_W, IDX_W)],
                    acc.at[or_v.at[buf].at[j]], ssem, add=True)

        def drain_scatter(buf):
            pltpu.make_async_copy(
                data_v.at[buf], acc.at[pl.ds(0, B_ROWS * IDX_W)],
                ssem).wait()

        def batch_pair(bb, carry):
            row0 = rule_base + (2 * bb) * B_ROWS
            do_batch(row0, 0)
            do_batch(row0 + B_ROWS, 1)
            fire_scatter(0)
            fire_scatter(1)
            drain_scatter(0)
            drain_scatter(1)
            return carry
        lax.fori_loop(0, NBATCH // 2, batch_pair, 0)
        plsc.subcore_barrier()

        for h in range(ACC_PT // INIT_ROWS):
            lo = sid * ACC_PT + h * INIT_ROWS
            pltpu.sync_copy(
                acc.at[pl.ds(lo, INIT_ROWS)],
                out_hbm.at[pl.ds(lo, INIT_ROWS), pl.ds(chunk * CW, CW)])
        plsc.subcore_barrier()


@functools.partial(
    pl.kernel,
    out_type=jax.ShapeDtypeStruct((N_PAD, CH), jnp.bfloat16),
    mesh=plsc.VectorSubcoreMesh(core_axis_name="c", subcore_axis_name="s",
                                num_cores=NCORE, num_subcores=NSUB),
    scratch_types=[
        pltpu.VMEM_SHARED((N_PAD, CW), jnp.bfloat16),
        pltpu.VMEM((CH,), jnp.bfloat16),
        pltpu.VMEM((INIT_ROWS, CW), jnp.bfloat16),
        pltpu.VMEM((2, B_ROWS, IDX_W), jnp.int32),
        pltpu.VMEM((2, B_ROWS, IDX_W), jnp.int32),
        pltpu.VMEM((2, B_ROWS * IDX_W, CW), jnp.bfloat16),
        pltpu.SemaphoreType.DMA,
        pltpu.SemaphoreType.DMA,
    ],
    compiler_params=pltpu.CompilerParams(use_tc_tiling_on_sc=False),
)
def _sc_scatter(tflat, gidx_all, orows, bias_in, out_hbm,
                acc, bias_v, init_v, gi_v, or_v, data_v, gsem, ssem):
    _sc_body(tflat, gidx_all, orows, bias_in, out_hbm,
             acc, bias_v, init_v, gi_v, or_v, data_v, gsem, ssem)


def kernel(coords, feats, rules, weight, bias):
    transformed = _transform(feats.astype(jnp.bfloat16),
                             weight.astype(jnp.bfloat16))
    tflat = transformed.reshape(K * N * NCHUNK, CW)

    offs = (jnp.arange(K, dtype=jnp.int32) * N).reshape(K, 1)
    gbase = ((rules[:, 0, :] + offs) * NCHUNK).reshape(NR_ROWS, IDX_W)
    gidx_all = gbase[None] + jnp.arange(NCHUNK, dtype=jnp.int32)[:, None, None]
    orows = rules[:, 1, :].reshape(NR_ROWS, IDX_W)

    out_pad = _sc_scatter(tflat, gidx_all, orows, bias.astype(jnp.bfloat16))
    return (coords, out_pad[:N].astype(jnp.float32))


# speedup vs baseline: 1.3994x; 1.3994x over previous
import functools

import jax
import jax.numpy as jnp
from jax import lax
from jax.experimental import pallas as pl
from jax.experimental.pallas import tpu as pltpu
from jax.experimental.pallas import tpu_sc as plsc

N = 100000
CH = 128
K = 9
R = 65536
NR = K * R
L = 16
NCORE = 2
NSUB = 16
NCHUNK = CH // L
PASSES = NCHUNK // NCORE

IDX_W = 128
NR_ROWS = NR // IDX_W
ROWS_PT = NR_ROWS // NSUB
B_ROWS = 4
NBATCH = ROWS_PT // B_ROWS
N_PAD = 100096
ACC_PT = N_PAD // NSUB
INIT_ROWS = 368

BR = 2000


def _mm_body(feats_ref, w_ref, out_ref):
    k = pl.program_id(1)
    out_ref[0] = jnp.dot(feats_ref[...], w_ref[k],
                         preferred_element_type=jnp.float32)


def _transform(feats_bf, weight_bf):
    return pl.pallas_call(
        _mm_body,
        grid=(N // BR, K),
        in_specs=[
            pl.BlockSpec((BR, CH), lambda i, k: (i, 0)),
            pl.BlockSpec((K, CH, CH), lambda i, k: (0, 0, 0)),
        ],
        out_specs=pl.BlockSpec((1, BR, CH), lambda i, k: (k, i, 0)),
        out_shape=jax.ShapeDtypeStruct((K, N, CH), jnp.float32),
    )(feats_bf, weight_bf)


def _sc_body(tflat, gidx_all, orows, bias_in, out_hbm,
             acc, bias_v, init_v, gi_v, or_v, data_v, gsem, ssem):
    cid = lax.axis_index("c")
    sid = lax.axis_index("s")
    rule_base = sid * ROWS_PT

    pltpu.sync_copy(bias_in, bias_v)

    for p in range(PASSES):
        chunk = cid * PASSES + p

        b0 = bias_v[pl.ds((0 * PASSES + p) * L, L)]
        b1 = bias_v[pl.ds((1 * PASSES + p) * L, L)]
        bias_c = jnp.where(cid == 0, b0, b1)

        def fill_row(i, carry):
            init_v[i] = bias_c
            return carry
        lax.fori_loop(0, INIT_ROWS, fill_row, 0)
        for h in range(ACC_PT // INIT_ROWS):
            pltpu.sync_copy(
                init_v, acc.at[pl.ds(sid * ACC_PT + h * INIT_ROWS, INIT_ROWS)])
        plsc.subcore_barrier()

        def do_batch(row_off, buf):
            pltpu.sync_copy(
                gidx_all.at[chunk].at[pl.ds(row_off, B_ROWS)], gi_v.at[buf])
            pltpu.sync_copy(orows.at[pl.ds(row_off, B_ROWS)], or_v.at[buf])
            for j in range(B_ROWS):
                pltpu.async_copy(
                    tflat.at[gi_v.at[buf].at[j]],
                    data_v.at[buf].at[pl.ds(j * IDX_W, IDX_W)], gsem)

        def fire_scatter(buf):
            pltpu.make_async_copy(
                tflat.at[pl.ds(0, B_ROWS * IDX_W)], data_v.at[buf],
                gsem).wait()
            for j in range(B_ROWS):
                pltpu.async_copy(
                    data_v.at[buf].at[pl.ds(j * IDX_W, IDX_W)],
                    acc.at[or_v.at[buf].at[j]], ssem, add=True)

        def drain_scatter(buf):
            pltpu.make_async_copy(
                data_v.at[buf], acc.at[pl.ds(0, B_ROWS * IDX_W)],
                ssem).wait()

        def batch_pair(bb, carry):
            row0 = rule_base + (2 * bb) * B_ROWS
            do_batch(row0, 0)
            do_batch(row0 + B_ROWS, 1)
            fire_scatter(0)
            fire_scatter(1)
            drain_scatter(0)
            drain_scatter(1)
            return carry
        lax.fori_loop(0, NBATCH // 2, batch_pair, 0)
        plsc.subcore_barrier()

        for h in range(ACC_PT // INIT_ROWS):
            lo = sid * ACC_PT + h * INIT_ROWS
            pltpu.sync_copy(
                acc.at[pl.ds(lo, INIT_ROWS)],
                out_hbm.at[pl.ds(lo, INIT_ROWS), pl.ds(chunk * L, L)])
        plsc.subcore_barrier()


@functools.partial(
    pl.kernel,
    out_type=jax.ShapeDtypeStruct((N_PAD, CH), jnp.float32),
    mesh=plsc.VectorSubcoreMesh(core_axis_name="c", subcore_axis_name="s",
                                num_cores=NCORE, num_subcores=NSUB),
    scratch_types=[
        pltpu.VMEM_SHARED((N_PAD, L), jnp.float32),
        pltpu.VMEM((CH,), jnp.float32),
        pltpu.VMEM((INIT_ROWS, L), jnp.float32),
        pltpu.VMEM((2, B_ROWS, IDX_W), jnp.int32),
        pltpu.VMEM((2, B_ROWS, IDX_W), jnp.int32),
        pltpu.VMEM((2, B_ROWS * IDX_W, L), jnp.float32),
        pltpu.SemaphoreType.DMA,
        pltpu.SemaphoreType.DMA,
    ],
    compiler_params=pltpu.CompilerParams(use_tc_tiling_on_sc=False),
)
def _sc_scatter(tflat, gidx_all, orows, bias_in, out_hbm,
                acc, bias_v, init_v, gi_v, or_v, data_v, gsem, ssem):
    _sc_body(tflat, gidx_all, orows, bias_in, out_hbm,
             acc, bias_v, init_v, gi_v, or_v, data_v, gsem, ssem)


def kernel(coords, feats, rules, weight, bias):
    transformed = _transform(feats.astype(jnp.bfloat16),
                             weight.astype(jnp.bfloat16))
    tflat = transformed.reshape(K * N * NCHUNK, L)

    offs = (jnp.arange(K, dtype=jnp.int32) * N).reshape(K, 1)
    gbase = ((rules[:, 0, :] + offs) * NCHUNK).reshape(NR_ROWS, IDX_W)
    gidx_all = gbase[None] + jnp.arange(NCHUNK, dtype=jnp.int32)[:, None, None]
    orows = rules[:, 1, :].reshape(NR_ROWS, IDX_W)

    out_pad = _sc_scatter(tflat, gidx_all, orows, bias)
    return (coords, out_pad[:N])


# speedup vs baseline: 1.5539x; 1.1104x over previous
import functools

import jax
import jax.numpy as jnp
from jax import lax
from jax.experimental import pallas as pl
from jax.experimental.pallas import tpu as pltpu
from jax.experimental.pallas import tpu_sc as plsc

N = 100000
CH = 128
K = 9
R = 65536
NR = K * R
L = 16
NCORE = 2
NSUB = 16
NCHUNK = CH // L
PASSES = NCHUNK // NCORE

IDX_W = 128
NR_ROWS = NR // IDX_W
ROWS_PT = NR_ROWS // NSUB
B_ROWS = 6
NBATCH = ROWS_PT // B_ROWS
N_PAD = 100096
ACC_PT = N_PAD // NSUB
INIT_ROWS = 184

BR = 2000


def _mm_body(feats_ref, w_ref, out_ref):
    k = pl.program_id(1)
    out_ref[0] = jnp.dot(feats_ref[...], w_ref[k],
                         preferred_element_type=jnp.float32)


def _transform(feats_bf, weight_bf):
    return pl.pallas_call(
        _mm_body,
        grid=(N // BR, K),
        in_specs=[
            pl.BlockSpec((BR, CH), lambda i, k: (i, 0)),
            pl.BlockSpec((K, CH, CH), lambda i, k: (0, 0, 0)),
        ],
        out_specs=pl.BlockSpec((1, BR, CH), lambda i, k: (k, i, 0)),
        out_shape=jax.ShapeDtypeStruct((K, N, CH), jnp.float32),
    )(feats_bf, weight_bf)


def _sc_body(tflat, gidx_all, orows, bias_in, out_hbm,
             acc, bias_v, init_v, gi_v, or_v, data_v, gsem, ssem):
    cid = lax.axis_index("c")
    sid = lax.axis_index("s")
    rule_base = sid * ROWS_PT

    pltpu.sync_copy(bias_in, bias_v)

    for p in range(PASSES):
        chunk = cid * PASSES + p

        b0 = bias_v[pl.ds((0 * PASSES + p) * L, L)]
        b1 = bias_v[pl.ds((1 * PASSES + p) * L, L)]
        bias_c = jnp.where(cid == 0, b0, b1)

        def fill_row(i, carry):
            init_v[i] = bias_c
            return carry
        lax.fori_loop(0, INIT_ROWS, fill_row, 0)
        for h in range(ACC_PT // INIT_ROWS):
            pltpu.sync_copy(
                init_v, acc.at[pl.ds(sid * ACC_PT + h * INIT_ROWS, INIT_ROWS)])
        plsc.subcore_barrier()

        def do_batch(row_off, buf):
            pltpu.sync_copy(
                gidx_all.at[chunk].at[pl.ds(row_off, B_ROWS)], gi_v.at[buf])
            pltpu.sync_copy(orows.at[pl.ds(row_off, B_ROWS)], or_v.at[buf])
            for j in range(B_ROWS):
                pltpu.async_copy(
                    tflat.at[gi_v.at[buf].at[j]],
                    data_v.at[buf].at[pl.ds(j * IDX_W, IDX_W)], gsem)

        def fire_scatter(buf):
            pltpu.make_async_copy(
                tflat.at[pl.ds(0, B_ROWS * IDX_W)], data_v.at[buf],
                gsem).wait()
            for j in range(B_ROWS):
                pltpu.async_copy(
                    data_v.at[buf].at[pl.ds(j * IDX_W, IDX_W)],
                    acc.at[or_v.at[buf].at[j]], ssem, add=True)

        def drain_scatter(buf):
            pltpu.make_async_copy(
                data_v.at[buf], acc.at[pl.ds(0, B_ROWS * IDX_W)],
                ssem).wait()

        def batch_pair(bb, carry):
            row0 = rule_base + (2 * bb) * B_ROWS
            do_batch(row0, 0)
            do_batch(row0 + B_ROWS, 1)
            fire_scatter(0)
            fire_scatter(1)
            drain_scatter(0)
            drain_scatter(1)
            return carry
        lax.fori_loop(0, NBATCH // 2, batch_pair, 0)
        plsc.subcore_barrier()

        for h in range(ACC_PT // INIT_ROWS):
            lo = sid * ACC_PT + h * INIT_ROWS
            pltpu.sync_copy(
                acc.at[pl.ds(lo, INIT_ROWS)],
                out_hbm.at[pl.ds(lo, INIT_ROWS), pl.ds(chunk * L, L)])
        plsc.subcore_barrier()


@functools.partial(
    pl.kernel,
    out_type=jax.ShapeDtypeStruct((N_PAD, CH), jnp.float32),
    mesh=plsc.VectorSubcoreMesh(core_axis_name="c", subcore_axis_name="s",
                                num_cores=NCORE, num_subcores=NSUB),
    scratch_types=[
        pltpu.VMEM_SHARED((N_PAD, L), jnp.float32),
        pltpu.VMEM((CH,), jnp.float32),
        pltpu.VMEM((INIT_ROWS, L), jnp.float32),
        pltpu.VMEM((2, B_ROWS, IDX_W), jnp.int32),
        pltpu.VMEM((2, B_ROWS, IDX_W), jnp.int32),
        pltpu.VMEM((2, B_ROWS * IDX_W, L), jnp.float32),
        pltpu.SemaphoreType.DMA,
        pltpu.SemaphoreType.DMA,
    ],
    compiler_params=pltpu.CompilerParams(use_tc_tiling_on_sc=False),
)
def _sc_scatter(tflat, gidx_all, orows, bias_in, out_hbm,
                acc, bias_v, init_v, gi_v, or_v, data_v, gsem, ssem):
    _sc_body(tflat, gidx_all, orows, bias_in, out_hbm,
             acc, bias_v, init_v, gi_v, or_v, data_v, gsem, ssem)


def kernel(coords, feats, rules, weight, bias):
    transformed = _transform(feats.astype(jnp.bfloat16),
                             weight.astype(jnp.bfloat16))
    tflat = transformed.reshape(K * N * NCHUNK, L)

    offs = (jnp.arange(K, dtype=jnp.int32) * N).reshape(K, 1)
    gbase = ((rules[:, 0, :] + offs) * NCHUNK).reshape(NR_ROWS, IDX_W)
    gidx_all = gbase[None] + jnp.arange(NCHUNK, dtype=jnp.int32)[:, None, None]
    orows = rules[:, 1, :].reshape(NR_ROWS, IDX_W)

    out_pad = _sc_scatter(tflat, gidx_all, orows, bias)
    return (coords, out_pad[:N])
